# Initial kernel scaffold; baseline (speedup 1.0000x reference)
import functools

import jax
import jax.numpy as jnp
from jax import lax
from jax.experimental import pallas as pl
from jax.experimental.pallas import tpu as pltpu
from jax.experimental.pallas import tpu_sc as plsc

N_USERS = 25000
N_ITEMS = 25000
N = N_USERS + N_ITEMS
D = 64
H = 32
E = 800000
N_LAYERS = 3

NS = 16
CHUNK = 128
K = 8
G = 49
ROWS_PER_TILE = G * K
ROWS128 = NS * ROWS_PER_TILE
E_PAD = ROWS128 * CHUNK
ROWS_OUT = N // NS
ZROWS = 125


def _sc_propagate(x_flat, rows2d, cols_lo, cols_hi, vals2d):
    mesh = plsc.VectorSubcoreMesh(core_axis_name="c", subcore_axis_name="s")
    y_t = jax.ShapeDtypeStruct((2 * N, H), jnp.float32)

    @functools.partial(
        pl.kernel,
        out_type=(y_t, y_t, y_t),
        mesh=mesh,
        scratch_types=[
            pltpu.VMEM((K, CHUNK), jnp.int32),
            pltpu.VMEM((K, CHUNK), jnp.int32),
            pltpu.VMEM((K, CHUNK), jnp.float32),
            pltpu.VMEM((K, CHUNK, H), jnp.float32),
            pltpu.VMEM((ZROWS, H), jnp.float32),
            pltpu.VMEM_SHARED((N, H), jnp.float32),
            pltpu.SemaphoreType.DMA,
        ],
    )
    def k(x_hbm, rows_hbm, clo_hbm, chi_hbm, vals_hbm,
          y1_hbm, y2_hbm, y3_hbm,
          rows_v, cols_v, vals_v, buf, zbuf, accum, gsem):
        c = lax.axis_index("c")
        s = lax.axis_index("s")

        z16 = jnp.zeros((16,), jnp.float32)

        @pl.loop(0, ZROWS)
        def _(i):
            zbuf[i, pl.ds(0, 16)] = z16
            zbuf[i, pl.ds(16, 16)] = z16

        def layer(src_hbm, dst_hbm):
            @pl.loop(0, ROWS_OUT // ZROWS)
            def _(i):
                pltpu.sync_copy(
                    zbuf, accum.at[pl.ds(s * ROWS_OUT + i * ZROWS, ZROWS)])
            plsc.subcore_barrier()

            @pl.loop(0, G)
            def _(g):
                r0 = s * ROWS_PER_TILE + g * K
                pltpu.sync_copy(rows_hbm.at[pl.ds(r0, K)], rows_v)
                pltpu.sync_copy(vals_hbm.at[pl.ds(r0, K)], vals_v)

                @pl.when(c == 0)
                def _():
                    pltpu.sync_copy(clo_hbm.at[pl.ds(r0, K)], cols_v)

                @pl.when(c != 0)
                def _():
                    pltpu.sync_copy(chi_hbm.at[pl.ds(r0, K)], cols_v)

                gathers = [
                    pltpu.async_copy(src_hbm.at[cols_v.at[j]], buf.at[j], gsem)
                    for j in range(K)
                ]
                for cp in gathers:
                    cp.wait()

                for j in range(K):
                    @pl.loop(0, CHUNK // 16)
                    def _(m, j=j):
                        e0 = m * 16
                        w16 = vals_v[j, pl.ds(e0, 16)]
                        for i in range(16):
                            w = jnp.take(w16, jnp.full((16,), i, jnp.int32),
                                         mode="promise_in_bounds")
                            lo = buf[j, e0 + i, pl.ds(0, 16)]
                            hi = buf[j, e0 + i, pl.ds(16, 16)]
                            buf[j, e0 + i, pl.ds(0, 16)] = lo * w
                            buf[j, e0 + i, pl.ds(16, 16)] = hi * w

                for j in range(K):
                    pltpu.sync_copy(buf.at[j], accum.at[rows_v.at[j]],
                                    add=True)

            plsc.subcore_barrier()
            pltpu.sync_copy(
                accum.at[pl.ds(s * ROWS_OUT, ROWS_OUT)],
                dst_hbm.at[pl.ds(c * N + s * ROWS_OUT, ROWS_OUT)])
            plsc.subcore_barrier()

        layer(x_hbm, y1_hbm)
        layer(y1_hbm, y2_hbm)
        layer(y2_hbm, y3_hbm)

    return k(x_flat, rows2d, cols_lo, cols_hi, vals2d)


def _tc_mean(x, y1, y2, y3):
    rows = 2 * N * H // 128
    xs = [a.reshape(rows, 128) for a in (x, y1, y2, y3)]
    br = 1000

    def body(x_ref, a_ref, b_ref, c_ref, o_ref):
        o_ref[...] = (x_ref[...] + a_ref[...] + b_ref[...] + c_ref[...]) * 0.25

    out = pl.pallas_call(
        body,
        out_shape=jax.ShapeDtypeStruct((rows, 128), jnp.float32),
        grid=(rows // br,),
        in_specs=[pl.BlockSpec((br, 128), lambda i: (i, 0))] * 4,
        out_specs=pl.BlockSpec((br, 128), lambda i: (i, 0)),
    )(*xs)
    return out.reshape(2 * N, H)


def kernel(user_emb, item_emb, edge_index, edge_values):
    all_emb = jnp.concatenate([user_emb, item_emb], axis=0)
    x_flat = jnp.concatenate([all_emb[:, :H], all_emb[:, H:]], axis=0)

    rows = edge_index[0]
    cols = edge_index[1]
    pad = E_PAD - E
    zi = jnp.zeros((pad,), jnp.int32)
    rows_p = jnp.concatenate([rows, zi]).reshape(ROWS128, CHUNK)
    cols_p = jnp.concatenate([cols, zi]).reshape(ROWS128, CHUNK)
    vals_p = jnp.concatenate(
        [edge_values, jnp.zeros((pad,), jnp.float32)]).reshape(ROWS128, CHUNK)
    cols_hi = cols_p + N

    y1, y2, y3 = _sc_propagate(x_flat, rows_p, cols_p, cols_hi, vals_p)
    m = _tc_mean(x_flat, y1, y2, y3)
    out = jnp.concatenate([m[:N], m[N:]], axis=1)
    return (out[:N_USERS], out[N_USERS:])


# speedup vs baseline: 6.8517x; 6.8517x over previous
import functools

import jax
import jax.numpy as jnp
from jax import lax
from jax.experimental import pallas as pl
from jax.experimental.pallas import tpu as pltpu
from jax.experimental.pallas import tpu_sc as plsc

N_USERS = 25000
N_ITEMS = 25000
N = N_USERS + N_ITEMS
D = 64
H = 32
E = 800000
N_LAYERS = 3

NS = 16
CHUNK = 128
K = 4
G = 98
ROWS_PER_TILE = G * K
ROWS128 = NS * ROWS_PER_TILE
E_PAD = ROWS128 * CHUNK
NP = 50048
ROWS_OUT = NP // NS


def _lane_broadcast(vec, i):
    idx = jnp.full((16, 1), i, jnp.int32)
    dnums = lax.GatherDimensionNumbers(
        offset_dims=(), collapsed_slice_dims=(0,), start_index_map=(0,))
    return lax.gather(vec, idx, dnums, slice_sizes=(1,),
                      mode=lax.GatherScatterMode.PROMISE_IN_BOUNDS)


def _sc_propagate(x_flat, rows2d, cols_lo, cols_hi, vals2d, zeros_hbm):
    mesh = plsc.VectorSubcoreMesh(core_axis_name="c", subcore_axis_name="s")
    y_t = jax.ShapeDtypeStruct((2 * NP, H), jnp.float32)

    @functools.partial(
        pl.kernel,
        out_type=(y_t, y_t, y_t),
        mesh=mesh,
        scratch_types=[
            pltpu.VMEM((K, CHUNK), jnp.int32),
            pltpu.VMEM((K, CHUNK), jnp.int32),
            pltpu.VMEM((K, CHUNK), jnp.float32),
            pltpu.VMEM((K, CHUNK, H), jnp.float32),
            pltpu.VMEM_SHARED((NP, H), jnp.float32),
            pltpu.SemaphoreType.DMA,
        ],
        compiler_params=pltpu.CompilerParams(use_tc_tiling_on_sc=False),
    )
    def k(x_hbm, rows_hbm, clo_hbm, chi_hbm, vals_hbm, z_hbm,
          y1_hbm, y2_hbm, y3_hbm,
          rows_v, cols_v, vals_v, buf, accum, gsem):
        c = lax.axis_index("c")
        s = lax.axis_index("s")

        def layer(src_hbm, dst_hbm):
            pltpu.sync_copy(z_hbm, accum.at[pl.ds(s * ROWS_OUT, ROWS_OUT)])
            plsc.subcore_barrier()

            @pl.loop(0, G)
            def _(g):
                r0 = s * ROWS_PER_TILE + g * K
                pltpu.sync_copy(rows_hbm.at[pl.ds(r0, K)], rows_v)
                pltpu.sync_copy(vals_hbm.at[pl.ds(r0, K)], vals_v)

                @pl.when(c == 0)
                def _():
                    pltpu.sync_copy(clo_hbm.at[pl.ds(r0, K)], cols_v)

                @pl.when(c != 0)
                def _():
                    pltpu.sync_copy(chi_hbm.at[pl.ds(r0, K)], cols_v)

                gathers = [
                    pltpu.async_copy(src_hbm.at[cols_v.at[j]], buf.at[j], gsem)
                    for j in range(K)
                ]
                for cp in gathers:
                    cp.wait()

                for j in range(K):
                    @pl.loop(0, CHUNK // 16)
                    def _(m, j=j):
                        e0 = m * 16
                        w16 = vals_v[j, pl.ds(e0, 16)]
                        for i in range(16):
                            w = _lane_broadcast(w16, i)
                            lo = buf[j, e0 + i, pl.ds(0, 16)]
                            hi = buf[j, e0 + i, pl.ds(16, 16)]
                            buf[j, e0 + i, pl.ds(0, 16)] = lo * w
                            buf[j, e0 + i, pl.ds(16, 16)] = hi * w

                for j in range(K):
                    pltpu.sync_copy(buf.at[j], accum.at[rows_v.at[j]],
                                    add=True)

            plsc.subcore_barrier()
            pltpu.sync_copy(
                accum.at[pl.ds(s * ROWS_OUT, ROWS_OUT)],
                dst_hbm.at[pl.ds(c * NP + s * ROWS_OUT, ROWS_OUT)])
            plsc.subcore_barrier()

        layer(x_hbm, y1_hbm)
        layer(y1_hbm, y2_hbm)
        layer(y2_hbm, y3_hbm)

    return k(x_flat, rows2d, cols_lo, cols_hi, vals2d, zeros_hbm)


def _tc_mean(x, y1, y2, y3):
    rows = 2 * NP * H // 128
    xs = [a.reshape(rows, 128) for a in (x, y1, y2, y3)]
    br = 3128

    def body(x_ref, a_ref, b_ref, c_ref, o_ref):
        o_ref[...] = (x_ref[...] + a_ref[...] + b_ref[...] + c_ref[...]) * 0.25

    out = pl.pallas_call(
        body,
        out_shape=jax.ShapeDtypeStruct((rows, 128), jnp.float32),
        grid=(rows // br,),
        in_specs=[pl.BlockSpec((br, 128), lambda i: (i, 0))] * 4,
        out_specs=pl.BlockSpec((br, 128), lambda i: (i, 0)),
    )(*xs)
    return out.reshape(2 * NP, H)


def kernel(user_emb, item_emb, edge_index, edge_values):
    all_emb = jnp.concatenate([user_emb, item_emb], axis=0)
    zrow = jnp.zeros((NP - N, H), jnp.float32)
    x_flat = jnp.concatenate([all_emb[:, :H], zrow, all_emb[:, H:], zrow],
                             axis=0)

    rows = edge_index[0]
    cols = edge_index[1]
    pad = E_PAD - E
    zi = jnp.zeros((pad,), jnp.int32)
    rows_p = jnp.concatenate([rows, zi]).reshape(ROWS128, CHUNK)
    cols_p = jnp.concatenate([cols, zi]).reshape(ROWS128, CHUNK)
    vals_p = jnp.concatenate(
        [edge_values, jnp.zeros((pad,), jnp.float32)]).reshape(ROWS128, CHUNK)
    cols_hi = cols_p + NP

    zeros_hbm = jnp.zeros((ROWS_OUT, H), jnp.float32)
    y1, y2, y3 = _sc_propagate(x_flat, rows_p, cols_p, cols_hi, vals_p,
                               zeros_hbm)
    m = _tc_mean(x_flat, y1, y2, y3)
    out = jnp.concatenate([m[:N], m[NP:NP + N]], axis=1)
    return (out[:N_USERS], out[N_USERS:])


# speedup vs baseline: 8.8212x; 1.2875x over previous
import functools

import jax
import jax.numpy as jnp
from jax import lax
from jax.experimental import pallas as pl
from jax.experimental.pallas import tpu as pltpu
from jax.experimental.pallas import tpu_sc as plsc

N_USERS = 25000
N_ITEMS = 25000
N = N_USERS + N_ITEMS
D = 64
H = 32
E = 800000
N_LAYERS = 3

NS = 16
CHUNK = 128
T = 392
NIDX = 4
NBUF = 2
ROWS_PER_TILE = T
ROWS128 = NS * ROWS_PER_TILE
E_PAD = ROWS128 * CHUNK
NP = 50048
ROWS_OUT = NP // NS


def _lane_broadcast(vec, i):
    idx = jnp.full((16, 1), i, jnp.int32)
    dnums = lax.GatherDimensionNumbers(
        offset_dims=(), collapsed_slice_dims=(0,), start_index_map=(0,))
    return lax.gather(vec, idx, dnums, slice_sizes=(1,),
                      mode=lax.GatherScatterMode.PROMISE_IN_BOUNDS)


def _sc_propagate(x_flat, rows2d, cols_lo, cols_hi, vals2d, zeros_hbm):
    mesh = plsc.VectorSubcoreMesh(core_axis_name="c", subcore_axis_name="s")
    y_t = jax.ShapeDtypeStruct((2 * NP, H), jnp.float32)

    @functools.partial(
        pl.kernel,
        out_type=(y_t, y_t, y_t),
        mesh=mesh,
        scratch_types=[
            pltpu.VMEM((NIDX, CHUNK), jnp.int32),
            pltpu.VMEM((NIDX, CHUNK), jnp.int32),
            pltpu.VMEM((NIDX, CHUNK), jnp.float32),
            pltpu.VMEM((NBUF, CHUNK, H), jnp.float32),
            pltpu.VMEM_SHARED((NP, H), jnp.float32),
            pltpu.SemaphoreType.DMA,
            pltpu.SemaphoreType.DMA,
            pltpu.SemaphoreType.DMA,
        ],
        compiler_params=pltpu.CompilerParams(use_tc_tiling_on_sc=False),
    )
    def k(x_hbm, rows_hbm, clo_hbm, chi_hbm, vals_hbm, z_hbm,
          y1_hbm, y2_hbm, y3_hbm,
          rows_v, cols_v, vals_v, buf, accum, isem, gsem, ssem):
        c = lax.axis_index("c")
        s = lax.axis_index("s")
        t0 = s * T

        def idx_dma(t):
            a = lax.rem(t, NIDX)
            r = t0 + t
            pltpu.async_copy(rows_hbm.at[pl.ds(r, 1)],
                             rows_v.at[pl.ds(a, 1)], isem)
            pltpu.async_copy(vals_hbm.at[pl.ds(r, 1)],
                             vals_v.at[pl.ds(a, 1)], isem)

            @pl.when(c == 0)
            def _():
                pltpu.async_copy(clo_hbm.at[pl.ds(r, 1)],
                                 cols_v.at[pl.ds(a, 1)], isem)

            @pl.when(c != 0)
            def _():
                pltpu.async_copy(chi_hbm.at[pl.ds(r, 1)],
                                 cols_v.at[pl.ds(a, 1)], isem)

        def wait_trio():
            pltpu.make_async_copy(rows_hbm.at[pl.ds(t0, 1)],
                                  rows_v.at[pl.ds(0, 1)], isem).wait()
            pltpu.make_async_copy(vals_hbm.at[pl.ds(t0, 1)],
                                  vals_v.at[pl.ds(0, 1)], isem).wait()
            pltpu.make_async_copy(rows_hbm.at[pl.ds(t0, 1)],
                                  cols_v.at[pl.ds(0, 1)], isem).wait()

        def layer(src_hbm, dst_hbm):
            pltpu.sync_copy(z_hbm, accum.at[pl.ds(s * ROWS_OUT, ROWS_OUT)])
            plsc.subcore_barrier()

            def fire_gather(t):
                a = lax.rem(t, NIDX)
                b = lax.rem(t, NBUF)
                pltpu.async_copy(src_hbm.at[cols_v.at[a]], buf.at[b], gsem)

            def wait_gather(t):
                a = lax.rem(t, NIDX)
                b = lax.rem(t, NBUF)
                pltpu.make_async_copy(src_hbm.at[cols_v.at[a]],
                                      buf.at[b], gsem).wait()

            def fire_scatter(t):
                a = lax.rem(t, NIDX)
                b = lax.rem(t, NBUF)
                pltpu.async_copy(buf.at[b], accum.at[rows_v.at[a]], ssem,
                                 add=True)

            def drain_scatter(t):
                a = lax.rem(t, NIDX)
                b = lax.rem(t, NBUF)
                pltpu.make_async_copy(buf.at[b],
                                      accum.at[rows_v.at[a]], ssem).wait()

            idx_dma(jnp.int32(0))
            idx_dma(jnp.int32(1))
            idx_dma(jnp.int32(2))
            wait_trio()
            fire_gather(jnp.int32(0))

            @pl.loop(0, T)
            def _(t):
                wait_gather(t)

                @pl.when(t > 0)
                def _():
                    drain_scatter(t - 1)

                @pl.when(t < T - 1)
                def _():
                    wait_trio()
                    fire_gather(t + 1)

                @pl.when(t < T - 3)
                def _():
                    idx_dma(t + 3)

                a = lax.rem(t, NIDX)
                b = lax.rem(t, NBUF)

                @pl.loop(0, CHUNK // 16)
                def _(m):
                    e0 = m * 16
                    w16 = vals_v[a, pl.ds(e0, 16)]
                    for i in range(16):
                        w = _lane_broadcast(w16, i)
                        lo = buf[b, e0 + i, pl.ds(0, 16)]
                        hi = buf[b, e0 + i, pl.ds(16, 16)]
                        buf[b, e0 + i, pl.ds(0, 16)] = lo * w
                        buf[b, e0 + i, pl.ds(16, 16)] = hi * w

                fire_scatter(t)

            drain_scatter(jnp.int32(T - 1))
            plsc.subcore_barrier()
            pltpu.sync_copy(
                accum.at[pl.ds(s * ROWS_OUT, ROWS_OUT)],
                dst_hbm.at[pl.ds(c * NP + s * ROWS_OUT, ROWS_OUT)])
            plsc.subcore_barrier()

        layer(x_hbm, y1_hbm)
        layer(y1_hbm, y2_hbm)
        layer(y2_hbm, y3_hbm)

    return k(x_flat, rows2d, cols_lo, cols_hi, vals2d, zeros_hbm)


def _tc_mean(x, y1, y2, y3):
    rows = 2 * NP * H // 128
    xs = [a.reshape(rows, 128) for a in (x, y1, y2, y3)]
    br = 3128

    def body(x_ref, a_ref, b_ref, c_ref, o_ref):
        o_ref[...] = (x_ref[...] + a_ref[...] + b_ref[...] + c_ref[...]) * 0.25

    out = pl.pallas_call(
        body,
        out_shape=jax.ShapeDtypeStruct((rows, 128), jnp.float32),
        grid=(rows // br,),
        in_specs=[pl.BlockSpec((br, 128), lambda i: (i, 0))] * 4,
        out_specs=pl.BlockSpec((br, 128), lambda i: (i, 0)),
    )(*xs)
    return out.reshape(2 * NP, H)


def kernel(user_emb, item_emb, edge_index, edge_values):
    all_emb = jnp.concatenate([user_emb, item_emb], axis=0)
    zrow = jnp.zeros((NP - N, H), jnp.float32)
    x_flat = jnp.concatenate([all_emb[:, :H], zrow, all_emb[:, H:], zrow],
                             axis=0)

    rows = edge_index[0]
    cols = edge_index[1]
    pad = E_PAD - E
    zi = jnp.zeros((pad,), jnp.int32)
    rows_p = jnp.concatenate([rows, zi]).reshape(ROWS128, CHUNK)
    cols_p = jnp.concatenate([cols, zi]).reshape(ROWS128, CHUNK)
    vals_p = jnp.concatenate(
        [edge_values, jnp.zeros((pad,), jnp.float32)]).reshape(ROWS128, CHUNK)
    cols_hi = cols_p + NP

    zeros_hbm = jnp.zeros((ROWS_OUT, H), jnp.float32)
    y1, y2, y3 = _sc_propagate(x_flat, rows_p, cols_p, cols_hi, vals_p,
                               zeros_hbm)
    m = _tc_mean(x_flat, y1, y2, y3)
    out = jnp.concatenate([m[:N], m[NP:NP + N]], axis=1)
    return (out[:N_USERS], out[N_USERS:])


# speedup vs baseline: 12.6075x; 1.4292x over previous
import functools

import jax
import jax.numpy as jnp
from jax import lax
from jax.experimental import pallas as pl
from jax.experimental.pallas import tpu as pltpu
from jax.experimental.pallas import tpu_sc as plsc

N_USERS = 25000
N_ITEMS = 25000
N = N_USERS + N_ITEMS
D = 64
H = 32
E = 800000
N_LAYERS = 3

NS = 16
CHUNK = 128
T = 392
NIDX = 8
NBUF = 4
GDEPTH = 3
IDEPTH = 6
ROWS_PER_TILE = T
ROWS128 = NS * ROWS_PER_TILE
E_PAD = ROWS128 * CHUNK
NP = 50048
ROWS_OUT = NP // NS


def _lane_broadcast(vec, i):
    idx = jnp.full((16, 1), i, jnp.int32)
    dnums = lax.GatherDimensionNumbers(
        offset_dims=(), collapsed_slice_dims=(0,), start_index_map=(0,))
    return lax.gather(vec, idx, dnums, slice_sizes=(1,),
                      mode=lax.GatherScatterMode.PROMISE_IN_BOUNDS)


def _sc_propagate(x_flat, rows2d, cols_lo, cols_hi, vals2d, zeros_hbm):
    mesh = plsc.VectorSubcoreMesh(core_axis_name="c", subcore_axis_name="s")
    y_t = jax.ShapeDtypeStruct((2 * NP, H), jnp.float32)

    @functools.partial(
        pl.kernel,
        out_type=(y_t, y_t, y_t),
        mesh=mesh,
        scratch_types=[
            pltpu.VMEM((NIDX, CHUNK), jnp.int32),
            pltpu.VMEM((NIDX, CHUNK), jnp.int32),
            pltpu.VMEM((NIDX, CHUNK), jnp.float32),
            pltpu.VMEM((NBUF, CHUNK, H), jnp.float32),
            pltpu.VMEM_SHARED((NP, H), jnp.float32),
            pltpu.SemaphoreType.DMA,
            pltpu.SemaphoreType.DMA((NBUF,)),
            pltpu.SemaphoreType.DMA,
        ],
        compiler_params=pltpu.CompilerParams(use_tc_tiling_on_sc=False),
    )
    def k(x_hbm, rows_hbm, clo_hbm, chi_hbm, vals_hbm, z_hbm,
          y1_hbm, y2_hbm, y3_hbm,
          rows_v, cols_v, vals_v, buf, accum, isem, gsem, ssem):
        c = lax.axis_index("c")
        s = lax.axis_index("s")
        t0 = s * T

        def idx_dma(t):
            a = lax.rem(t, NIDX)
            r = t0 + t
            pltpu.async_copy(rows_hbm.at[pl.ds(r, 1)],
                             rows_v.at[pl.ds(a, 1)], isem)
            pltpu.async_copy(vals_hbm.at[pl.ds(r, 1)],
                             vals_v.at[pl.ds(a, 1)], isem)

            @pl.when(c == 0)
            def _():
                pltpu.async_copy(clo_hbm.at[pl.ds(r, 1)],
                                 cols_v.at[pl.ds(a, 1)], isem)

            @pl.when(c != 0)
            def _():
                pltpu.async_copy(chi_hbm.at[pl.ds(r, 1)],
                                 cols_v.at[pl.ds(a, 1)], isem)

        def wait_trio():
            pltpu.make_async_copy(rows_hbm.at[pl.ds(t0, 1)],
                                  rows_v.at[pl.ds(0, 1)], isem).wait()
            pltpu.make_async_copy(vals_hbm.at[pl.ds(t0, 1)],
                                  vals_v.at[pl.ds(0, 1)], isem).wait()
            pltpu.make_async_copy(rows_hbm.at[pl.ds(t0, 1)],
                                  cols_v.at[pl.ds(0, 1)], isem).wait()

        def layer(src_hbm, dst_hbm):
            pltpu.sync_copy(z_hbm, accum.at[pl.ds(s * ROWS_OUT, ROWS_OUT)])
            plsc.subcore_barrier()

            def fire_gather(t):
                a = lax.rem(t, NIDX)
                b = lax.rem(t, NBUF)
                pltpu.async_copy(src_hbm.at[cols_v.at[a]], buf.at[b],
                                 gsem.at[b])

            def wait_gather(t):
                a = lax.rem(t, NIDX)
                b = lax.rem(t, NBUF)
                pltpu.make_async_copy(src_hbm.at[cols_v.at[a]],
                                      buf.at[b], gsem.at[b]).wait()

            def fire_scatter(t):
                a = lax.rem(t, NIDX)
                b = lax.rem(t, NBUF)
                pltpu.async_copy(buf.at[b], accum.at[rows_v.at[a]], ssem,
                                 add=True)

            def drain_scatter(t):
                a = lax.rem(t, NIDX)
                b = lax.rem(t, NBUF)
                pltpu.make_async_copy(buf.at[b],
                                      accum.at[rows_v.at[a]], ssem).wait()

            for u in range(IDEPTH):
                idx_dma(jnp.int32(u))
            for u in range(GDEPTH):
                wait_trio()
                fire_gather(jnp.int32(u))

            @pl.loop(0, T)
            def _(t):
                wait_gather(t)

                @pl.when(t > 0)
                def _():
                    drain_scatter(t - 1)

                @pl.when(t < T - GDEPTH)
                def _():
                    wait_trio()
                    fire_gather(t + GDEPTH)

                @pl.when(t < T - IDEPTH)
                def _():
                    idx_dma(t + IDEPTH)

                a = lax.rem(t, NIDX)
                b = lax.rem(t, NBUF)

                @pl.loop(0, CHUNK // 16)
                def _(m):
                    e0 = m * 16
                    w16 = vals_v[a, pl.ds(e0, 16)]
                    for i in range(16):
                        w = _lane_broadcast(w16, i)
                        lo = buf[b, e0 + i, pl.ds(0, 16)]
                        hi = buf[b, e0 + i, pl.ds(16, 16)]
                        buf[b, e0 + i, pl.ds(0, 16)] = lo * w
                        buf[b, e0 + i, pl.ds(16, 16)] = hi * w

                fire_scatter(t)

            drain_scatter(jnp.int32(T - 1))
            plsc.subcore_barrier()
            pltpu.sync_copy(
                accum.at[pl.ds(s * ROWS_OUT, ROWS_OUT)],
                dst_hbm.at[pl.ds(c * NP + s * ROWS_OUT, ROWS_OUT)])
            plsc.subcore_barrier()

        layer(x_hbm, y1_hbm)
        layer(y1_hbm, y2_hbm)
        layer(y2_hbm, y3_hbm)

    return k(x_flat, rows2d, cols_lo, cols_hi, vals2d, zeros_hbm)


def _tc_mean(x, y1, y2, y3):
    rows = 2 * NP * H // 128
    xs = [a.reshape(rows, 128) for a in (x, y1, y2, y3)]
    br = 3128

    def body(x_ref, a_ref, b_ref, c_ref, o_ref):
        o_ref[...] = (x_ref[...] + a_ref[...] + b_ref[...] + c_ref[...]) * 0.25

    out = pl.pallas_call(
        body,
        out_shape=jax.ShapeDtypeStruct((rows, 128), jnp.float32),
        grid=(rows // br,),
        in_specs=[pl.BlockSpec((br, 128), lambda i: (i, 0))] * 4,
        out_specs=pl.BlockSpec((br, 128), lambda i: (i, 0)),
    )(*xs)
    return out.reshape(2 * NP, H)


def kernel(user_emb, item_emb, edge_index, edge_values):
    all_emb = jnp.concatenate([user_emb, item_emb], axis=0)
    zrow = jnp.zeros((NP - N, H), jnp.float32)
    x_flat = jnp.concatenate([all_emb[:, :H], zrow, all_emb[:, H:], zrow],
                             axis=0)

    rows = edge_index[0]
    cols = edge_index[1]
    pad = E_PAD - E
    zi = jnp.zeros((pad,), jnp.int32)
    rows_p = jnp.concatenate([rows, zi]).reshape(ROWS128, CHUNK)
    cols_p = jnp.concatenate([cols, zi]).reshape(ROWS128, CHUNK)
    vals_p = jnp.concatenate(
        [edge_values, jnp.zeros((pad,), jnp.float32)]).reshape(ROWS128, CHUNK)
    cols_hi = cols_p + NP

    zeros_hbm = jnp.zeros((ROWS_OUT, H), jnp.float32)
    y1, y2, y3 = _sc_propagate(x_flat, rows_p, cols_p, cols_hi, vals_p,
                               zeros_hbm)
    m = _tc_mean(x_flat, y1, y2, y3)
    out = jnp.concatenate([m[:N], m[NP:NP + N]], axis=1)
    return (out[:N_USERS], out[N_USERS:])


# speedup vs baseline: 12.6675x; 1.0048x over previous
import functools

import jax
import jax.numpy as jnp
from jax import lax
from jax.experimental import pallas as pl
from jax.experimental.pallas import tpu as pltpu
from jax.experimental.pallas import tpu_sc as plsc

N_USERS = 25000
N_ITEMS = 25000
N = N_USERS + N_ITEMS
D = 64
H = 32
E = 800000
N_LAYERS = 3

NS = 16
CHUNK = 128
T = 392
G = 8
NG = T // G
NIDXG = 3
NBUF = 4
GDEPTH = 3
ROWS_PER_TILE = T
ROWS128 = NS * ROWS_PER_TILE
E_PAD = ROWS128 * CHUNK
NP = 50048
ROWS_OUT = NP // NS


def _lane_broadcast(vec, i):
    idx = jnp.full((16, 1), i, jnp.int32)
    dnums = lax.GatherDimensionNumbers(
        offset_dims=(), collapsed_slice_dims=(0,), start_index_map=(0,))
    return lax.gather(vec, idx, dnums, slice_sizes=(1,),
                      mode=lax.GatherScatterMode.PROMISE_IN_BOUNDS)


def _sc_propagate(x_flat, rows2d, cols_lo, cols_hi, vals2d, zeros_hbm):
    mesh = plsc.VectorSubcoreMesh(core_axis_name="c", subcore_axis_name="s")
    y_t = jax.ShapeDtypeStruct((2 * NP, H), jnp.float32)

    @functools.partial(
        pl.kernel,
        out_type=(y_t, y_t, y_t),
        mesh=mesh,
        scratch_types=[
            pltpu.VMEM((NIDXG * G, CHUNK), jnp.int32),
            pltpu.VMEM((NIDXG * G, CHUNK), jnp.int32),
            pltpu.VMEM((NIDXG * G, CHUNK), jnp.float32),
            pltpu.VMEM((NBUF, CHUNK, H), jnp.float32),
            pltpu.VMEM_SHARED((NP, H), jnp.float32),
            pltpu.SemaphoreType.DMA,
            pltpu.SemaphoreType.DMA((NBUF,)),
            pltpu.SemaphoreType.DMA,
        ],
        compiler_params=pltpu.CompilerParams(use_tc_tiling_on_sc=False),
    )
    def k(x_hbm, rows_hbm, clo_hbm, chi_hbm, vals_hbm, z_hbm,
          y1_hbm, y2_hbm, y3_hbm,
          rows_v, cols_v, vals_v, buf, accum, isem, gsem, ssem):
        c = lax.axis_index("c")
        s = lax.axis_index("s")
        t0 = s * T

        def fire_group(g):
            slot = lax.rem(g, NIDXG) * G
            r = t0 + g * G
            pltpu.async_copy(rows_hbm.at[pl.ds(r, G)],
                             rows_v.at[pl.ds(slot, G)], isem)
            pltpu.async_copy(vals_hbm.at[pl.ds(r, G)],
                             vals_v.at[pl.ds(slot, G)], isem)

            @pl.when(c == 0)
            def _():
                pltpu.async_copy(clo_hbm.at[pl.ds(r, G)],
                                 cols_v.at[pl.ds(slot, G)], isem)

            @pl.when(c != 0)
            def _():
                pltpu.async_copy(chi_hbm.at[pl.ds(r, G)],
                                 cols_v.at[pl.ds(slot, G)], isem)

        def wait_group():
            for _ in range(3):
                pltpu.make_async_copy(rows_hbm.at[pl.ds(t0, G)],
                                      rows_v.at[pl.ds(0, G)], isem).wait()

        def layer(src_hbm, dst_hbm):
            pltpu.sync_copy(z_hbm, accum.at[pl.ds(s * ROWS_OUT, ROWS_OUT)])
            plsc.subcore_barrier()

            def fire_gather(row, b):
                pltpu.async_copy(src_hbm.at[cols_v.at[row]], buf.at[b],
                                 gsem.at[b])

            def wait_gather(row, b):
                pltpu.make_async_copy(src_hbm.at[cols_v.at[row]],
                                      buf.at[b], gsem.at[b]).wait()

            def fire_scatter(row, b):
                pltpu.async_copy(buf.at[b], accum.at[rows_v.at[row]], ssem,
                                 add=True)

            def drain_scatter(row, b):
                pltpu.make_async_copy(buf.at[b],
                                      accum.at[rows_v.at[row]], ssem).wait()

            fire_group(jnp.int32(0))
            fire_group(jnp.int32(1))
            fire_group(jnp.int32(2))
            wait_group()
            for u in range(GDEPTH):
                fire_gather(jnp.int32(u), jnp.int32(u))

            @pl.loop(0, NG)
            def _(g):
                base = lax.rem(g, NIDXG) * G
                basen = lax.rem(g + 1, NIDXG) * G

                @pl.loop(0, G)
                def _(ti):
                    t = g * G + ti
                    b = lax.rem(ti, NBUF)
                    row = base + ti
                    wait_gather(row, b)

                    @pl.when(t > 0)
                    def _():
                        drain_scatter(row, b)

                    @pl.when((ti == 1) & (g >= 1) & (g + 2 < NG))
                    def _():
                        fire_group(g + 2)

                    @pl.when((ti == G - GDEPTH) & (g + 1 < NG))
                    def _():
                        wait_group()

                    @pl.when(t + GDEPTH < T)
                    def _():
                        bf = lax.rem(ti + GDEPTH, NBUF)
                        rowf = jnp.where(ti < G - GDEPTH,
                                         base + ti + GDEPTH,
                                         basen + ti + GDEPTH - G)
                        fire_gather(rowf, bf)

                    @pl.loop(0, CHUNK // 16)
                    def _(m):
                        e0 = m * 16
                        w16 = vals_v[row, pl.ds(e0, 16)]
                        for i in range(16):
                            w = _lane_broadcast(w16, i)
                            lo = buf[b, e0 + i, pl.ds(0, 16)]
                            hi = buf[b, e0 + i, pl.ds(16, 16)]
                            buf[b, e0 + i, pl.ds(0, 16)] = lo * w
                            buf[b, e0 + i, pl.ds(16, 16)] = hi * w

                    fire_scatter(row, b)

            drain_scatter(jnp.int32(0), jnp.int32((T - 1) % NBUF))
            plsc.subcore_barrier()
            pltpu.sync_copy(
                accum.at[pl.ds(s * ROWS_OUT, ROWS_OUT)],
                dst_hbm.at[pl.ds(c * NP + s * ROWS_OUT, ROWS_OUT)])
            plsc.subcore_barrier()

        layer(x_hbm, y1_hbm)
        layer(y1_hbm, y2_hbm)
        layer(y2_hbm, y3_hbm)

    return k(x_flat, rows2d, cols_lo, cols_hi, vals2d, zeros_hbm)


def _tc_mean(x, y1, y2, y3):
    rows = 2 * NP * H // 128
    xs = [a.reshape(rows, 128) for a in (x, y1, y2, y3)]
    br = 3128

    def body(x_ref, a_ref, b_ref, c_ref, o_ref):
        o_ref[...] = (x_ref[...] + a_ref[...] + b_ref[...] + c_ref[...]) * 0.25

    out = pl.pallas_call(
        body,
        out_shape=jax.ShapeDtypeStruct((rows, 128), jnp.float32),
        grid=(rows // br,),
        in_specs=[pl.BlockSpec((br, 128), lambda i: (i, 0))] * 4,
        out_specs=pl.BlockSpec((br, 128), lambda i: (i, 0)),
    )(*xs)
    return out.reshape(2 * NP, H)


def kernel(user_emb, item_emb, edge_index, edge_values):
    all_emb = jnp.concatenate([user_emb, item_emb], axis=0)
    zrow = jnp.zeros((NP - N, H), jnp.float32)
    x_flat = jnp.concatenate([all_emb[:, :H], zrow, all_emb[:, H:], zrow],
                             axis=0)

    rows = edge_index[0]
    cols = edge_index[1]
    pad = E_PAD - E
    zi = jnp.zeros((pad,), jnp.int32)
    rows_p = jnp.concatenate([rows, zi]).reshape(ROWS128, CHUNK)
    cols_p = jnp.concatenate([cols, zi]).reshape(ROWS128, CHUNK)
    vals_p = jnp.concatenate(
        [edge_values, jnp.zeros((pad,), jnp.float32)]).reshape(ROWS128, CHUNK)
    cols_hi = cols_p + NP

    zeros_hbm = jnp.zeros((ROWS_OUT, H), jnp.float32)
    y1, y2, y3 = _sc_propagate(x_flat, rows_p, cols_p, cols_hi, vals_p,
                               zeros_hbm)
    m = _tc_mean(x_flat, y1, y2, y3)
    out = jnp.concatenate([m[:N], m[NP:NP + N]], axis=1)
    return (out[:N_USERS], out[N_USERS:])


# speedup vs baseline: 12.7634x; 1.0076x over previous
import functools

import jax
import jax.numpy as jnp
from jax import lax
from jax.experimental import pallas as pl
from jax.experimental.pallas import tpu as pltpu
from jax.experimental.pallas import tpu_sc as plsc

N_USERS = 25000
N_ITEMS = 25000
N = N_USERS + N_ITEMS
D = 64
H = 32
E = 800000
N_LAYERS = 3

NS = 16
CHUNK = 128
T = 392
G = 8
NG = T // G
NIDXG = 3
NBUF = 4
GDEPTH = 3
ROWS_PER_TILE = T
ROWS128 = NS * ROWS_PER_TILE
E_PAD = ROWS128 * CHUNK
NP = 50048
ROWS_OUT = NP // NS


def _lane_broadcast(vec, i):
    idx = jnp.full((16, 1), i, jnp.int32)
    dnums = lax.GatherDimensionNumbers(
        offset_dims=(), collapsed_slice_dims=(0,), start_index_map=(0,))
    return lax.gather(vec, idx, dnums, slice_sizes=(1,),
                      mode=lax.GatherScatterMode.PROMISE_IN_BOUNDS)


def _sc_propagate(x_flat, rows2d, cols_lo, cols_hi, vals2d, zeros_hbm):
    mesh = plsc.VectorSubcoreMesh(core_axis_name="c", subcore_axis_name="s")
    y_t = jax.ShapeDtypeStruct((2 * NP, H), jnp.float32)

    @functools.partial(
        pl.kernel,
        out_type=(y_t, y_t, y_t),
        mesh=mesh,
        scratch_types=[
            pltpu.VMEM((NIDXG * G, CHUNK), jnp.int32),
            pltpu.VMEM((NIDXG * G, CHUNK), jnp.int32),
            pltpu.VMEM((NIDXG * G, CHUNK), jnp.float32),
            pltpu.VMEM((NBUF, CHUNK, H), jnp.float32),
            pltpu.VMEM_SHARED((NP, H), jnp.float32),
            pltpu.SemaphoreType.DMA,
            pltpu.SemaphoreType.DMA((NBUF,)),
            pltpu.SemaphoreType.DMA,
        ],
        compiler_params=pltpu.CompilerParams(use_tc_tiling_on_sc=False),
    )
    def k(x_hbm, rows_hbm, clo_hbm, chi_hbm, vals_hbm, z_hbm,
          y1_hbm, y2_hbm, y3_hbm,
          rows_v, cols_v, vals_v, buf, accum, isem, gsem, ssem):
        c = lax.axis_index("c")
        s = lax.axis_index("s")
        t0 = s * T

        def fire_group(g):
            slot = lax.rem(g, NIDXG) * G
            r = t0 + g * G
            pltpu.async_copy(rows_hbm.at[pl.ds(r, G)],
                             rows_v.at[pl.ds(slot, G)], isem)
            pltpu.async_copy(vals_hbm.at[pl.ds(r, G)],
                             vals_v.at[pl.ds(slot, G)], isem)

            @pl.when(c == 0)
            def _():
                pltpu.async_copy(clo_hbm.at[pl.ds(r, G)],
                                 cols_v.at[pl.ds(slot, G)], isem)

            @pl.when(c != 0)
            def _():
                pltpu.async_copy(chi_hbm.at[pl.ds(r, G)],
                                 cols_v.at[pl.ds(slot, G)], isem)

        def wait_group():
            for _ in range(3):
                pltpu.make_async_copy(rows_hbm.at[pl.ds(t0, G)],
                                      rows_v.at[pl.ds(0, G)], isem).wait()

        def layer(src_hbm, dst_hbm):
            pltpu.sync_copy(z_hbm, accum.at[pl.ds(s * ROWS_OUT, ROWS_OUT)])
            plsc.subcore_barrier()

            def fire_gather(row, b):
                pltpu.async_copy(src_hbm.at[cols_v.at[row]], buf.at[b],
                                 gsem.at[b])

            def wait_gather(row, b):
                pltpu.make_async_copy(src_hbm.at[cols_v.at[row]],
                                      buf.at[b], gsem.at[b]).wait()

            def fire_scatter(row, b):
                pltpu.async_copy(buf.at[b], accum.at[rows_v.at[row]], ssem,
                                 add=True)

            def drain_scatter(row, b):
                pltpu.make_async_copy(buf.at[b],
                                      accum.at[rows_v.at[row]], ssem).wait()

            fire_group(jnp.int32(0))
            fire_group(jnp.int32(1))
            fire_group(jnp.int32(2))
            wait_group()
            for u in range(GDEPTH):
                fire_gather(jnp.int32(u), jnp.int32(u))

            @pl.loop(0, NG)
            def _(g):
                base = lax.rem(g, NIDXG) * G
                basen = lax.rem(g + 1, NIDXG) * G

                for ti in range(G):
                    b = ti % NBUF
                    row = base + ti

                    wait_gather(row, b)

                    if ti == 0:
                        @pl.when(g > 0)
                        def _():
                            drain_scatter(row, b)
                    else:
                        drain_scatter(row, b)

                    if ti == 1:
                        @pl.when((g >= 1) & (g + 2 < NG))
                        def _():
                            fire_group(g + 2)

                    bf = (ti + GDEPTH) % NBUF
                    if ti < G - GDEPTH:
                        fire_gather(base + ti + GDEPTH, bf)
                    else:
                        @pl.when(g + 1 < NG)
                        def _():
                            if ti == G - GDEPTH:
                                wait_group()
                            fire_gather(basen + ti + GDEPTH - G, bf)

                    @pl.loop(0, CHUNK // 16)
                    def _(m):
                        e0 = m * 16
                        w16 = vals_v[row, pl.ds(e0, 16)]
                        for i in range(16):
                            w = _lane_broadcast(w16, i)
                            lo = buf[b, e0 + i, pl.ds(0, 16)]
                            hi = buf[b, e0 + i, pl.ds(16, 16)]
                            buf[b, e0 + i, pl.ds(0, 16)] = lo * w
                            buf[b, e0 + i, pl.ds(16, 16)] = hi * w

                    fire_scatter(row, b)

            drain_scatter(jnp.int32(0), jnp.int32((T - 1) % NBUF))
            plsc.subcore_barrier()
            pltpu.sync_copy(
                accum.at[pl.ds(s * ROWS_OUT, ROWS_OUT)],
                dst_hbm.at[pl.ds(c * NP + s * ROWS_OUT, ROWS_OUT)])
            plsc.subcore_barrier()

        layer(x_hbm, y1_hbm)
        layer(y1_hbm, y2_hbm)
        layer(y2_hbm, y3_hbm)

    return k(x_flat, rows2d, cols_lo, cols_hi, vals2d, zeros_hbm)


def _tc_mean(x, y1, y2, y3):
    rows = 2 * NP * H // 128
    xs = [a.reshape(rows, 128) for a in (x, y1, y2, y3)]
    br = 3128

    def body(x_ref, a_ref, b_ref, c_ref, o_ref):
        o_ref[...] = (x_ref[...] + a_ref[...] + b_ref[...] + c_ref[...]) * 0.25

    out = pl.pallas_call(
        body,
        out_shape=jax.ShapeDtypeStruct((rows, 128), jnp.float32),
        grid=(rows // br,),
        in_specs=[pl.BlockSpec((br, 128), lambda i: (i, 0))] * 4,
        out_specs=pl.BlockSpec((br, 128), lambda i: (i, 0)),
    )(*xs)
    return out.reshape(2 * NP, H)


def kernel(user_emb, item_emb, edge_index, edge_values):
    all_emb = jnp.concatenate([user_emb, item_emb], axis=0)
    zrow = jnp.zeros((NP - N, H), jnp.float32)
    x_flat = jnp.concatenate([all_emb[:, :H], zrow, all_emb[:, H:], zrow],
                             axis=0)

    rows = edge_index[0]
    cols = edge_index[1]
    pad = E_PAD - E
    zi = jnp.zeros((pad,), jnp.int32)
    rows_p = jnp.concatenate([rows, zi]).reshape(ROWS128, CHUNK)
    cols_p = jnp.concatenate([cols, zi]).reshape(ROWS128, CHUNK)
    vals_p = jnp.concatenate(
        [edge_values, jnp.zeros((pad,), jnp.float32)]).reshape(ROWS128, CHUNK)
    cols_hi = cols_p + NP

    zeros_hbm = jnp.zeros((ROWS_OUT, H), jnp.float32)
    y1, y2, y3 = _sc_propagate(x_flat, rows_p, cols_p, cols_hi, vals_p,
                               zeros_hbm)
    m = _tc_mean(x_flat, y1, y2, y3)
    out = jnp.concatenate([m[:N], m[NP:NP + N]], axis=1)
    return (out[:N_USERS], out[N_USERS:])
